# baseline (speedup 1.0000x reference)
import functools

import jax
import jax.numpy as jnp
from jax import lax
from jax.experimental import pallas as pl
from jax.experimental.pallas import tpu as pltpu
from jax.experimental.pallas import tpu_sc as plsc

N, E, D, H = 10000, 160000, 256, 256
NPAD = 10240
EPAD = 163840
CHUNK = 64
SCH = 128
NSC = 2
NSUB = 16
HHALF = 128
SLAB = NPAD // NSUB
RB = 2000
GRID = N // RB


def _fill_f32(ref, value, nrows):
    ncol = ref.shape[1] // 16

    def body(i, _):
        for k in range(ncol):
            ref[i, pl.ds(k * 16, 16)] = jnp.full((16,), value, jnp.float32)
        return 0

    lax.fori_loop(0, nrows, body, 0)


def _tc_onehot_body(dst_r, oh_r):
    lane = lax.broadcasted_iota(jnp.int32, oh_r.shape, 1)
    dmod = lax.rem(dst_r[...], jnp.int32(128))
    oh_r[...] = (dmod == lane).astype(jnp.float32)


def _tc_onehot(dst2):
    rbe = 8192
    return pl.pallas_call(
        _tc_onehot_body,
        grid=(EPAD // rbe,),
        in_specs=[pl.BlockSpec((rbe, 1), lambda i: (i, 0))],
        out_specs=pl.BlockSpec((rbe, 128), lambda i: (i, 0)),
        out_shape=jax.ShapeDtypeStruct((EPAD, 128), jnp.float32),
    )(dst2)


NQ = NPAD // 128


def _sc_degree(oh, dst3):
    mesh = plsc.VectorSubcoreMesh(
        core_axis_name="c", subcore_axis_name="s", num_cores=NSC,
        num_subcores=NSUB)
    nchunk = EPAD // (NSC * NSUB) // CHUNK
    per_w = nchunk * CHUNK

    nstage = nchunk // 2

    @functools.partial(
        pl.kernel,
        out_type=jax.ShapeDtypeStruct((NSC, NQ, 128), jnp.float32),
        mesh=mesh,
        scratch_types=[
            pltpu.VMEM((nstage, CHUNK), jnp.int32),
            pltpu.VMEM((CHUNK, 128), jnp.float32),
            pltpu.VMEM_SHARED((NQ, 128), jnp.float32),
        ],
    )
    def deg_kernel(oh_hbm, dst_hbm, out_hbm, idx_v, buf, acc):
        c = lax.axis_index("c")
        s = lax.axis_index("s")
        w = c * NSUB + s

        @pl.when(s == 0)
        def _():
            _fill_f32(buf, 0.0, 16)
            for t in range(NQ // 16):
                pltpu.sync_copy(buf.at[pl.ds(0, 16)], acc.at[pl.ds(t * 16, 16)])

        plsc.subcore_barrier()

        for m in range(2):
            pltpu.sync_copy(dst_hbm.at[w, pl.ds(m * nstage, nstage)], idx_v)

            def shift(j, _):
                for k in range(CHUNK // 16):
                    sl = pl.ds(k * 16, 16)
                    idx_v[j, sl] = lax.shift_right_logical(idx_v[j, sl], 7)
                return 0

            lax.fori_loop(0, nstage, shift, 0)
            base = w * per_w + m * nstage * CHUNK

            def body(j, _):
                pltpu.sync_copy(oh_hbm.at[pl.ds(base + j * CHUNK, CHUNK)], buf)
                pltpu.sync_copy(buf, acc.at[idx_v.at[j]], add=True)
                return 0

            lax.fori_loop(0, nstage, body, 0)
        plsc.subcore_barrier()

        @pl.when(s == 0)
        def _():
            pltpu.sync_copy(acc, out_hbm.at[c])

    return deg_kernel(oh, dst3)


def _tc_prep_body(x_r, w1_r, w2_r, deg_r, h_r, whh_r, bhh_r, y_r, gh_r):
    wsum = w1_r[...] + w2_r[...]
    xw = lax.dot_general(x_r[...], wsum, (((1,), (1,)), ((), ())),
                         preferred_element_type=jnp.float32)
    dinv = lax.rsqrt(deg_r[...] + 1.0)
    y = xw * dinv
    y_r[0] = y[:, :HHALF]
    y_r[1] = y[:, HHALF:]
    gh_r[...] = lax.dot_general(h_r[...], whh_r[...], (((1,), (1,)), ((), ())),
                                preferred_element_type=jnp.float32) + bhh_r[...]


def _tc_prep(x, W1, W2, degf, h2, W_hh, b_hh2):
    return pl.pallas_call(
        _tc_prep_body,
        grid=(GRID,),
        in_specs=[
            pl.BlockSpec((RB, D), lambda i: (i, 0)),
            pl.BlockSpec((H, D), lambda i: (0, 0)),
            pl.BlockSpec((H, D), lambda i: (0, 0)),
            pl.BlockSpec((RB, 1), lambda i: (i, 0)),
            pl.BlockSpec((1, H), lambda i: (0, 0)),
            pl.BlockSpec((3 * H, H), lambda i: (0, 0)),
            pl.BlockSpec((1, 3 * H), lambda i: (0, 0)),
        ],
        out_specs=[
            pl.BlockSpec((NSC, RB, HHALF), lambda i: (0, i, 0)),
            pl.BlockSpec((1, 3 * H), lambda i: (0, 0)),
        ],
        out_shape=[
            jax.ShapeDtypeStruct((NSC, NPAD, HHALF), jnp.float32),
            jax.ShapeDtypeStruct((1, 3 * H), jnp.float32),
        ],
    )(x, W1, W2, degf, h2, W_hh, b_hh2)


def _sc_scatter(yflat, srcT, dstT):
    mesh = plsc.VectorSubcoreMesh(
        core_axis_name="c", subcore_axis_name="s", num_cores=NSC,
        num_subcores=NSUB)
    nchunk = EPAD // NSUB // SCH
    nstage = nchunk // 8

    @functools.partial(
        pl.kernel,
        out_type=jax.ShapeDtypeStruct((NSC, NPAD, HHALF), jnp.float32),
        mesh=mesh,
        scratch_types=[
            pltpu.VMEM((nstage, SCH), jnp.int32),
            pltpu.VMEM((nstage, SCH), jnp.int32),
            pltpu.VMEM((SCH, HHALF), jnp.float32),
            pltpu.VMEM((SCH, HHALF), jnp.float32),
            pltpu.VMEM_SHARED((NPAD, HHALF), jnp.float32),
            pltpu.SemaphoreType.DMA,
            pltpu.SemaphoreType.DMA,
        ],
    )
    def scat_kernel(y_hbm, src_hbm, dst_hbm, out_hbm,
                    isrc, idst, rows0, rows1, acc, sem0, sem1):
        c = lax.axis_index("c")
        s = lax.axis_index("s")
        _fill_f32(rows0, 0.0, SCH)
        for t in range(SLAB // SCH):
            pltpu.sync_copy(rows0, acc.at[pl.ds(s * SLAB + t * SCH, SCH)])
        plsc.subcore_barrier()
        off = c * NPAD
        bufs = ((rows0, sem0), (rows1, sem1))

        for m in range(8):
            pltpu.sync_copy(src_hbm.at[s, m], isrc)
            pltpu.sync_copy(dst_hbm.at[s, m], idst)

            def add_off(j, _):
                for k in range(SCH // 16):
                    sl = pl.ds(k * 16, 16)
                    isrc[j, sl] = isrc[j, sl] + off
                return 0

            lax.fori_loop(0, nstage, add_off, 0)

            for b in range(2):
                rows, sem = bufs[b]
                pltpu.async_copy(y_hbm.at[isrc.at[b]], rows, sem)

            def body(i, _):
                for b in range(2):
                    rows, sem = bufs[b]
                    j = 2 * i + b
                    pltpu.make_async_copy(y_hbm.at[isrc.at[j]], rows,
                                          sem).wait()
                    pltpu.sync_copy(rows, acc.at[idst.at[j]], add=True)

                    @pl.when(j + 2 < nstage)
                    def _():
                        pltpu.async_copy(y_hbm.at[isrc.at[j + 2]], rows, sem)

                return 0

            lax.fori_loop(0, nstage // 2, body, 0)
        plsc.subcore_barrier()
        pltpu.sync_copy(acc.at[pl.ds(s * SLAB, SLAB)],
                        out_hbm.at[c, pl.ds(s * SLAB, SLAB)])

    return scat_kernel(yflat, srcT, dstT)


def _tc_final_body(S_r, y_r, deg_r, b1_r, b2_r, wih_r, bih_r, gh_r, h_r,
                   hn_r, psum_r, psq_r):
    dinv = lax.rsqrt(deg_r[...] + 1.0)
    g = jnp.concatenate([S_r[0] + y_r[0], S_r[1] + y_r[1]], axis=1)
    g = jax.nn.relu(g * dinv + (b1_r[...] + b2_r[...]))
    gi = lax.dot_general(g, wih_r[...], (((1,), (1,)), ((), ())),
                         preferred_element_type=jnp.float32) + bih_r[...]
    gh = gh_r[...]
    r = jax.nn.sigmoid(gi[:, :H] + gh[:, :H])
    z = jax.nn.sigmoid(gi[:, H:2 * H] + gh[:, H:2 * H])
    n = jnp.tanh(gi[:, 2 * H:] + r * gh[:, 2 * H:])
    h_new = (1.0 - z) * n + z * h_r[...]
    hn_r[...] = h_new
    psum_r[...] = jnp.sum(h_new, axis=0, keepdims=True)[None]
    psq_r[...] = jnp.sum(h_new * h_new, axis=0, keepdims=True)[None]


def _tc_final(S, y, degf, b1_2, b2_2, W_ih, b_ih2, gh, h2):
    return pl.pallas_call(
        _tc_final_body,
        grid=(GRID,),
        in_specs=[
            pl.BlockSpec((NSC, RB, HHALF), lambda i: (0, i, 0)),
            pl.BlockSpec((NSC, RB, HHALF), lambda i: (0, i, 0)),
            pl.BlockSpec((RB, 1), lambda i: (i, 0)),
            pl.BlockSpec((1, H), lambda i: (0, 0)),
            pl.BlockSpec((1, H), lambda i: (0, 0)),
            pl.BlockSpec((3 * H, H), lambda i: (0, 0)),
            pl.BlockSpec((1, 3 * H), lambda i: (0, 0)),
            pl.BlockSpec((1, 3 * H), lambda i: (0, 0)),
            pl.BlockSpec((1, H), lambda i: (0, 0)),
        ],
        out_specs=[
            pl.BlockSpec((RB, H), lambda i: (i, 0)),
            pl.BlockSpec((1, 1, H), lambda i: (i, 0, 0)),
            pl.BlockSpec((1, 1, H), lambda i: (i, 0, 0)),
        ],
        out_shape=[
            jax.ShapeDtypeStruct((N, H), jnp.float32),
            jax.ShapeDtypeStruct((GRID, 1, H), jnp.float32),
            jax.ShapeDtypeStruct((GRID, 1, H), jnp.float32),
        ],
    )(S, y, degf, b1_2, b2_2, W_ih, b_ih2, gh, h2)


def _tc_norm_body(hn_r, psum_r, psq_r, gamma_r, beta_r, out_r):
    tot = jnp.sum(psum_r[...], axis=0)
    tot2 = jnp.sum(psq_r[...], axis=0)
    mean = tot * (1.0 / N)
    var = tot2 * (1.0 / N) - mean * mean
    inv = gamma_r[...] * lax.rsqrt(var + 1e-5)
    out_r[...] = (hn_r[...] - mean) * inv + beta_r[...]


def _tc_norm(hn, psum, psq, gamma2, beta2):
    return pl.pallas_call(
        _tc_norm_body,
        grid=(GRID,),
        in_specs=[
            pl.BlockSpec((RB, H), lambda i: (i, 0)),
            pl.BlockSpec((GRID, 1, H), lambda i: (0, 0, 0)),
            pl.BlockSpec((GRID, 1, H), lambda i: (0, 0, 0)),
            pl.BlockSpec((1, H), lambda i: (0, 0)),
            pl.BlockSpec((1, H), lambda i: (0, 0)),
        ],
        out_specs=pl.BlockSpec((RB, H), lambda i: (i, 0)),
        out_shape=jax.ShapeDtypeStruct((N, H), jnp.float32),
    )(hn, psum, psq, gamma2, beta2)


def kernel(x, edge_index, h, W1, b1, W2, b2, W_ih, b_ih, W_hh, b_hh,
           gamma, beta):
    src = edge_index[0]
    dst = edge_index[1]
    pad = jnp.full((EPAD - E,), N, jnp.int32)
    srcp = jnp.concatenate([src, pad])
    dstp = jnp.concatenate([dst, pad])

    oh = _tc_onehot(dstp.reshape(EPAD, 1))
    degH = _sc_degree(oh, dstp.reshape(NSC * NSUB, -1, CHUNK))
    degf = (degH[0] + degH[1]).reshape(NPAD, 1)

    h2 = h.reshape(1, H)
    y, gh = _tc_prep(x, W1, W2, degf, h2, W_hh, b_hh.reshape(1, 3 * H))

    S = _sc_scatter(y.reshape(NSC * NPAD, HHALF),
                    srcp.reshape(NSUB, 8, -1, SCH),
                    dstp.reshape(NSUB, 8, -1, SCH))

    hn, psum, psq = _tc_final(S, y, degf, b1.reshape(1, H), b2.reshape(1, H),
                              W_ih, b_ih.reshape(1, 3 * H), gh, h2)
    return _tc_norm(hn, psum, psq, gamma.reshape(1, H), beta.reshape(1, H))


# speedup vs baseline: 1.1059x; 1.1059x over previous
import functools

import jax
import jax.numpy as jnp
from jax import lax
from jax.experimental import pallas as pl
from jax.experimental.pallas import tpu as pltpu
from jax.experimental.pallas import tpu_sc as plsc

N, E, D, H = 10000, 160000, 256, 256
NPAD = 10240
EPAD = 163840
CHUNK = 64
SCH = 64
NSC = 2
NSUB = 16
HHALF = 128
SLAB = NPAD // NSUB
RB = 2000
GRID = N // RB


def _fill_f32(ref, value, nrows):
    ncol = ref.shape[1] // 16

    def body(i, _):
        for k in range(ncol):
            ref[i, pl.ds(k * 16, 16)] = jnp.full((16,), value, jnp.float32)
        return 0

    lax.fori_loop(0, nrows, body, 0)


NQ = NPAD // 128


def _sc_degree(eye, dst3):
    mesh = plsc.VectorSubcoreMesh(
        core_axis_name="c", subcore_axis_name="s", num_cores=NSC,
        num_subcores=NSUB)
    nchunk = EPAD // (NSC * NSUB) // CHUNK

    @functools.partial(
        pl.kernel,
        out_type=jax.ShapeDtypeStruct((NSC, NQ, 128), jnp.float32),
        mesh=mesh,
        scratch_types=[
            pltpu.VMEM((nchunk, CHUNK), jnp.int32),
            pltpu.VMEM((nchunk, CHUNK), jnp.int32),
            pltpu.VMEM((CHUNK, 128), jnp.float32),
            pltpu.VMEM_SHARED((128, 128), jnp.float32),
            pltpu.VMEM_SHARED((NQ, 128), jnp.float32),
            pltpu.SemaphoreType.DMA,
        ],
    )
    def deg_kernel(eye_hbm, dst_hbm, out_hbm, idq_v, idm_v, buf, eye_sp,
                   acc, sem):
        c = lax.axis_index("c")
        s = lax.axis_index("s")
        w = c * NSUB + s

        @pl.when(s == 0)
        def _():
            pltpu.sync_copy(eye_hbm, eye_sp)
            _fill_f32(buf, 0.0, 16)
            for t in range(NQ // 16):
                pltpu.sync_copy(buf.at[pl.ds(0, 16)], acc.at[pl.ds(t * 16, 16)])

        plsc.subcore_barrier()
        pltpu.sync_copy(dst_hbm.at[w], idq_v)
        pltpu.sync_copy(dst_hbm.at[w], idm_v)

        def prep(j, _):
            for k in range(CHUNK // 16):
                sl = pl.ds(k * 16, 16)
                idm_v[j, sl] = jnp.bitwise_and(idm_v[j, sl], 127)
                idq_v[j, sl] = lax.shift_right_logical(idq_v[j, sl], 7)
            return 0

        lax.fori_loop(0, nchunk, prep, 0)

        def body(j, _):
            pltpu.async_copy(eye_sp.at[idm_v.at[j]], buf, sem).wait()
            pltpu.sync_copy(buf, acc.at[idq_v.at[j]], add=True)
            return 0

        lax.fori_loop(0, nchunk, body, 0)
        plsc.subcore_barrier()

        @pl.when(s == 0)
        def _():
            pltpu.sync_copy(acc, out_hbm.at[c])

    return deg_kernel(eye, dst3)


def _tc_prep_body(x_r, w1_r, w2_r, deg_r, h_r, whh_r, bhh_r, y_r, gh_r):
    wsum = w1_r[...] + w2_r[...]
    xw = lax.dot_general(x_r[...], wsum, (((1,), (1,)), ((), ())),
                         preferred_element_type=jnp.float32)
    dinv = lax.rsqrt(deg_r[...] + 1.0)
    y = xw * dinv
    y_r[0] = y[:, :HHALF]
    y_r[1] = y[:, HHALF:]
    gh_r[...] = lax.dot_general(h_r[...], whh_r[...], (((1,), (1,)), ((), ())),
                                preferred_element_type=jnp.float32) + bhh_r[...]


def _tc_prep(x, W1, W2, degf, h2, W_hh, b_hh2):
    return pl.pallas_call(
        _tc_prep_body,
        grid=(GRID,),
        in_specs=[
            pl.BlockSpec((RB, D), lambda i: (i, 0)),
            pl.BlockSpec((H, D), lambda i: (0, 0)),
            pl.BlockSpec((H, D), lambda i: (0, 0)),
            pl.BlockSpec((RB, 1), lambda i: (i, 0)),
            pl.BlockSpec((1, H), lambda i: (0, 0)),
            pl.BlockSpec((3 * H, H), lambda i: (0, 0)),
            pl.BlockSpec((1, 3 * H), lambda i: (0, 0)),
        ],
        out_specs=[
            pl.BlockSpec((NSC, RB, HHALF), lambda i: (0, i, 0)),
            pl.BlockSpec((1, 3 * H), lambda i: (0, 0)),
        ],
        out_shape=[
            jax.ShapeDtypeStruct((NSC, NPAD, HHALF), jnp.float32),
            jax.ShapeDtypeStruct((1, 3 * H), jnp.float32),
        ],
    )(x, W1, W2, degf, h2, W_hh, b_hh2)


def _sc_scatter(yflat, srcT, dstT):
    mesh = plsc.VectorSubcoreMesh(
        core_axis_name="c", subcore_axis_name="s", num_cores=NSC,
        num_subcores=NSUB)
    nchunk = EPAD // NSUB // SCH
    nstage = nchunk // 2

    @functools.partial(
        pl.kernel,
        out_type=jax.ShapeDtypeStruct((NSC, NPAD, HHALF), jnp.float32),
        mesh=mesh,
        scratch_types=[
            pltpu.VMEM((nstage, SCH), jnp.int32),
            pltpu.VMEM((nstage, SCH), jnp.int32),
            pltpu.VMEM((SCH, HHALF), jnp.float32),
            pltpu.VMEM((SCH, HHALF), jnp.float32),
            pltpu.VMEM_SHARED((NPAD, HHALF), jnp.float32),
            pltpu.SemaphoreType.DMA,
            pltpu.SemaphoreType.DMA,
        ],
    )
    def scat_kernel(y_hbm, src_hbm, dst_hbm, out_hbm,
                    isrc, idst, rows0, rows1, acc, sem0, sem1):
        c = lax.axis_index("c")
        s = lax.axis_index("s")
        _fill_f32(rows0, 0.0, SCH)
        for t in range(SLAB // SCH):
            pltpu.sync_copy(rows0, acc.at[pl.ds(s * SLAB + t * SCH, SCH)])
        plsc.subcore_barrier()
        off = c * NPAD
        bufs = ((rows0, sem0), (rows1, sem1))

        for m in range(2):
            pltpu.sync_copy(src_hbm.at[s, m], isrc)
            pltpu.sync_copy(dst_hbm.at[s, m], idst)

            def add_off(j, _):
                for k in range(SCH // 16):
                    sl = pl.ds(k * 16, 16)
                    isrc[j, sl] = isrc[j, sl] + off
                return 0

            lax.fori_loop(0, nstage, add_off, 0)

            for b in range(2):
                rows, sem = bufs[b]
                pltpu.async_copy(y_hbm.at[isrc.at[b]], rows, sem)

            def body(i, _):
                for b in range(2):
                    rows, sem = bufs[b]
                    j = 2 * i + b
                    pltpu.make_async_copy(y_hbm.at[isrc.at[j]], rows,
                                          sem).wait()
                    pltpu.sync_copy(rows, acc.at[idst.at[j]], add=True)

                    @pl.when(j + 2 < nstage)
                    def _():
                        pltpu.async_copy(y_hbm.at[isrc.at[j + 2]], rows, sem)

                return 0

            lax.fori_loop(0, nstage // 2, body, 0)
        plsc.subcore_barrier()
        pltpu.sync_copy(acc.at[pl.ds(s * SLAB, SLAB)],
                        out_hbm.at[c, pl.ds(s * SLAB, SLAB)])

    return scat_kernel(yflat, srcT, dstT)


def _tc_final_body(S_r, y_r, deg_r, b1_r, b2_r, wih_r, bih_r, gh_r, h_r,
                   hn_r, psum_r, psq_r):
    dinv = lax.rsqrt(deg_r[...] + 1.0)
    g = jnp.concatenate([S_r[0] + y_r[0], S_r[1] + y_r[1]], axis=1)
    g = jax.nn.relu(g * dinv + (b1_r[...] + b2_r[...]))
    gi = lax.dot_general(g, wih_r[...], (((1,), (1,)), ((), ())),
                         preferred_element_type=jnp.float32) + bih_r[...]
    gh = gh_r[...]
    r = jax.nn.sigmoid(gi[:, :H] + gh[:, :H])
    z = jax.nn.sigmoid(gi[:, H:2 * H] + gh[:, H:2 * H])
    n = jnp.tanh(gi[:, 2 * H:] + r * gh[:, 2 * H:])
    h_new = (1.0 - z) * n + z * h_r[...]
    hn_r[...] = h_new
    psum_r[...] = jnp.sum(h_new, axis=0, keepdims=True)[None]
    psq_r[...] = jnp.sum(h_new * h_new, axis=0, keepdims=True)[None]


def _tc_final(S, y, degf, b1_2, b2_2, W_ih, b_ih2, gh, h2):
    return pl.pallas_call(
        _tc_final_body,
        grid=(GRID,),
        in_specs=[
            pl.BlockSpec((NSC, RB, HHALF), lambda i: (0, i, 0)),
            pl.BlockSpec((NSC, RB, HHALF), lambda i: (0, i, 0)),
            pl.BlockSpec((RB, 1), lambda i: (i, 0)),
            pl.BlockSpec((1, H), lambda i: (0, 0)),
            pl.BlockSpec((1, H), lambda i: (0, 0)),
            pl.BlockSpec((3 * H, H), lambda i: (0, 0)),
            pl.BlockSpec((1, 3 * H), lambda i: (0, 0)),
            pl.BlockSpec((1, 3 * H), lambda i: (0, 0)),
            pl.BlockSpec((1, H), lambda i: (0, 0)),
        ],
        out_specs=[
            pl.BlockSpec((RB, H), lambda i: (i, 0)),
            pl.BlockSpec((1, 1, H), lambda i: (i, 0, 0)),
            pl.BlockSpec((1, 1, H), lambda i: (i, 0, 0)),
        ],
        out_shape=[
            jax.ShapeDtypeStruct((N, H), jnp.float32),
            jax.ShapeDtypeStruct((GRID, 1, H), jnp.float32),
            jax.ShapeDtypeStruct((GRID, 1, H), jnp.float32),
        ],
    )(S, y, degf, b1_2, b2_2, W_ih, b_ih2, gh, h2)


def _tc_norm_body(hn_r, psum_r, psq_r, gamma_r, beta_r, out_r):
    tot = jnp.sum(psum_r[...], axis=0)
    tot2 = jnp.sum(psq_r[...], axis=0)
    mean = tot * (1.0 / N)
    var = tot2 * (1.0 / N) - mean * mean
    inv = gamma_r[...] * lax.rsqrt(var + 1e-5)
    out_r[...] = (hn_r[...] - mean) * inv + beta_r[...]


def _tc_norm(hn, psum, psq, gamma2, beta2):
    return pl.pallas_call(
        _tc_norm_body,
        grid=(GRID,),
        in_specs=[
            pl.BlockSpec((RB, H), lambda i: (i, 0)),
            pl.BlockSpec((GRID, 1, H), lambda i: (0, 0, 0)),
            pl.BlockSpec((GRID, 1, H), lambda i: (0, 0, 0)),
            pl.BlockSpec((1, H), lambda i: (0, 0)),
            pl.BlockSpec((1, H), lambda i: (0, 0)),
        ],
        out_specs=pl.BlockSpec((RB, H), lambda i: (i, 0)),
        out_shape=jax.ShapeDtypeStruct((N, H), jnp.float32),
    )(hn, psum, psq, gamma2, beta2)


def kernel(x, edge_index, h, W1, b1, W2, b2, W_ih, b_ih, W_hh, b_hh,
           gamma, beta):
    src = edge_index[0]
    dst = edge_index[1]
    pad = jnp.full((EPAD - E,), N, jnp.int32)
    srcp = jnp.concatenate([src, pad])
    dstp = jnp.concatenate([dst, pad])

    eye = jnp.eye(128, dtype=jnp.float32)
    degH = _sc_degree(eye, dstp.reshape(NSC * NSUB, -1, CHUNK))
    degf = (degH[0] + degH[1]).reshape(NPAD, 1)

    h2 = h.reshape(1, H)
    y, gh = _tc_prep(x, W1, W2, degf, h2, W_hh, b_hh.reshape(1, 3 * H))

    S = _sc_scatter(y.reshape(NSC * NPAD, HHALF),
                    srcp.reshape(NSUB, 2, -1, SCH),
                    dstp.reshape(NSUB, 2, -1, SCH))

    hn, psum, psq = _tc_final(S, y, degf, b1.reshape(1, H), b2.reshape(1, H),
                              W_ih, b_ih.reshape(1, 3 * H), gh, h2)
    return _tc_norm(hn, psum, psq, gamma.reshape(1, H), beta.reshape(1, H))


# speedup vs baseline: 1.1759x; 1.0633x over previous
import functools

import jax
import jax.numpy as jnp
from jax import lax
from jax.experimental import pallas as pl
from jax.experimental.pallas import tpu as pltpu
from jax.experimental.pallas import tpu_sc as plsc

N, E, D, H = 10000, 160000, 256, 256
NPAD = 10240
EPAD = 163840
CHUNK = 64
SCH = 64
NSC = 2
NSUB = 16
HHALF = 128
SLAB = NPAD // NSUB
RB = 2000
GRID = N // RB


def _fill_f32(ref, value, nrows):
    ncol = ref.shape[1] // 16

    def body(i, _):
        for k in range(ncol):
            ref[i, pl.ds(k * 16, 16)] = jnp.full((16,), value, jnp.float32)
        return 0

    lax.fori_loop(0, nrows, body, 0)


NQ = NPAD // 128


def _sc_degree(eye, dst3):
    mesh = plsc.VectorSubcoreMesh(
        core_axis_name="c", subcore_axis_name="s", num_cores=NSC,
        num_subcores=NSUB)
    nchunk = EPAD // (NSC * NSUB) // CHUNK

    @functools.partial(
        pl.kernel,
        out_type=jax.ShapeDtypeStruct((NSC, NQ, 128), jnp.float32),
        mesh=mesh,
        scratch_types=[
            pltpu.VMEM((nchunk, CHUNK), jnp.int32),
            pltpu.VMEM((nchunk, CHUNK), jnp.int32),
            pltpu.VMEM((CHUNK, 128), jnp.float32),
            pltpu.VMEM((CHUNK, 128), jnp.float32),
            pltpu.VMEM_SHARED((128, 128), jnp.float32),
            pltpu.VMEM_SHARED((NQ, 128), jnp.float32),
            pltpu.SemaphoreType.DMA,
            pltpu.SemaphoreType.DMA,
        ],
    )
    def deg_kernel(eye_hbm, dst_hbm, out_hbm, idq_v, idm_v, buf0, buf1,
                   eye_sp, acc, sem0, sem1):
        c = lax.axis_index("c")
        s = lax.axis_index("s")
        w = c * NSUB + s

        @pl.when(s == 0)
        def _():
            pltpu.sync_copy(eye_hbm, eye_sp)
            _fill_f32(buf0, 0.0, 16)
            for t in range(NQ // 16):
                pltpu.sync_copy(buf0.at[pl.ds(0, 16)],
                                acc.at[pl.ds(t * 16, 16)])

        plsc.subcore_barrier()
        pltpu.sync_copy(dst_hbm.at[w], idq_v)
        pltpu.sync_copy(dst_hbm.at[w], idm_v)

        def prep(j, _):
            for k in range(CHUNK // 16):
                sl = pl.ds(k * 16, 16)
                idm_v[j, sl] = jnp.bitwise_and(idm_v[j, sl], 127)
                idq_v[j, sl] = lax.shift_right_logical(idq_v[j, sl], 7)
            return 0

        lax.fori_loop(0, nchunk, prep, 0)
        bufs = ((buf0, sem0), (buf1, sem1))
        for b in range(2):
            pltpu.async_copy(eye_sp.at[idm_v.at[b]], bufs[b][0], bufs[b][1])

        def body(i, _):
            for b in range(2):
                buf, sem = bufs[b]
                j = 2 * i + b
                pltpu.make_async_copy(eye_sp.at[idm_v.at[j]], buf, sem).wait()
                pltpu.sync_copy(buf, acc.at[idq_v.at[j]], add=True)

                @pl.when(j + 2 < nchunk)
                def _():
                    pltpu.async_copy(eye_sp.at[idm_v.at[j + 2]], buf, sem)

            return 0

        lax.fori_loop(0, nchunk // 2, body, 0)
        plsc.subcore_barrier()

        @pl.when(s == 0)
        def _():
            pltpu.sync_copy(acc, out_hbm.at[c])

    return deg_kernel(eye, dst3)


def _tc_prep_body(x_r, w1_r, w2_r, deg_r, h_r, whh_r, bhh_r, y_r, gh_r):
    wsum = w1_r[...] + w2_r[...]
    xw = lax.dot_general(x_r[...], wsum, (((1,), (1,)), ((), ())),
                         preferred_element_type=jnp.float32)
    dinv = lax.rsqrt(deg_r[...] + 1.0)
    y = xw * dinv
    y_r[0] = y[:, :HHALF]
    y_r[1] = y[:, HHALF:]
    gh_r[...] = lax.dot_general(h_r[...], whh_r[...], (((1,), (1,)), ((), ())),
                                preferred_element_type=jnp.float32) + bhh_r[...]


def _tc_prep(x, W1, W2, degf, h2, W_hh, b_hh2):
    return pl.pallas_call(
        _tc_prep_body,
        grid=(GRID,),
        in_specs=[
            pl.BlockSpec((RB, D), lambda i: (i, 0)),
            pl.BlockSpec((H, D), lambda i: (0, 0)),
            pl.BlockSpec((H, D), lambda i: (0, 0)),
            pl.BlockSpec((RB, 1), lambda i: (i, 0)),
            pl.BlockSpec((1, H), lambda i: (0, 0)),
            pl.BlockSpec((3 * H, H), lambda i: (0, 0)),
            pl.BlockSpec((1, 3 * H), lambda i: (0, 0)),
        ],
        out_specs=[
            pl.BlockSpec((NSC, RB, HHALF), lambda i: (0, i, 0)),
            pl.BlockSpec((1, 3 * H), lambda i: (0, 0)),
        ],
        out_shape=[
            jax.ShapeDtypeStruct((NSC, NPAD, HHALF), jnp.float32),
            jax.ShapeDtypeStruct((1, 3 * H), jnp.float32),
        ],
    )(x, W1, W2, degf, h2, W_hh, b_hh2)


def _sc_scatter(yflat, srcT, dstT):
    mesh = plsc.VectorSubcoreMesh(
        core_axis_name="c", subcore_axis_name="s", num_cores=NSC,
        num_subcores=NSUB)
    nchunk = EPAD // NSUB // SCH
    nstage = nchunk // 2

    @functools.partial(
        pl.kernel,
        out_type=jax.ShapeDtypeStruct((NSC, NPAD, HHALF), jnp.float32),
        mesh=mesh,
        scratch_types=[
            pltpu.VMEM((nstage, SCH), jnp.int32),
            pltpu.VMEM((nstage, SCH), jnp.int32),
            pltpu.VMEM((SCH, HHALF), jnp.float32),
            pltpu.VMEM((SCH, HHALF), jnp.float32),
            pltpu.VMEM_SHARED((NPAD, HHALF), jnp.float32),
            pltpu.SemaphoreType.DMA,
            pltpu.SemaphoreType.DMA,
        ],
    )
    def scat_kernel(y_hbm, src_hbm, dst_hbm, out_hbm,
                    isrc, idst, rows0, rows1, acc, sem0, sem1):
        c = lax.axis_index("c")
        s = lax.axis_index("s")
        _fill_f32(rows0, 0.0, SCH)
        for t in range(SLAB // SCH):
            pltpu.sync_copy(rows0, acc.at[pl.ds(s * SLAB + t * SCH, SCH)])
        plsc.subcore_barrier()
        off = c * NPAD
        bufs = ((rows0, sem0), (rows1, sem1))

        for m in range(2):
            pltpu.sync_copy(src_hbm.at[s, m], isrc)
            pltpu.sync_copy(dst_hbm.at[s, m], idst)

            def add_off(j, _):
                for k in range(SCH // 16):
                    sl = pl.ds(k * 16, 16)
                    isrc[j, sl] = isrc[j, sl] + off
                return 0

            lax.fori_loop(0, nstage, add_off, 0)

            for b in range(2):
                rows, sem = bufs[b]
                pltpu.async_copy(y_hbm.at[isrc.at[b]], rows, sem)

            def body(i, _):
                for b in range(2):
                    rows, sem = bufs[b]
                    j = 2 * i + b
                    pltpu.make_async_copy(y_hbm.at[isrc.at[j]], rows,
                                          sem).wait()
                    pltpu.sync_copy(rows, acc.at[idst.at[j]], add=True)

                    @pl.when(j + 2 < nstage)
                    def _():
                        pltpu.async_copy(y_hbm.at[isrc.at[j + 2]], rows, sem)

                return 0

            lax.fori_loop(0, nstage // 2, body, 0)
        plsc.subcore_barrier()
        pltpu.sync_copy(acc.at[pl.ds(s * SLAB, SLAB)],
                        out_hbm.at[c, pl.ds(s * SLAB, SLAB)])

    return scat_kernel(yflat, srcT, dstT)


def _tc_final_body(S_r, y_r, deg_r, b1_r, b2_r, wih_r, bih_r, gh_r, h_r,
                   hn_r, psum_r, psq_r):
    dinv = lax.rsqrt(deg_r[...] + 1.0)
    g = jnp.concatenate([S_r[0] + y_r[0], S_r[1] + y_r[1]], axis=1)
    g = jax.nn.relu(g * dinv + (b1_r[...] + b2_r[...]))
    gi = lax.dot_general(g, wih_r[...], (((1,), (1,)), ((), ())),
                         preferred_element_type=jnp.float32) + bih_r[...]
    gh = gh_r[...]
    r = jax.nn.sigmoid(gi[:, :H] + gh[:, :H])
    z = jax.nn.sigmoid(gi[:, H:2 * H] + gh[:, H:2 * H])
    n = jnp.tanh(gi[:, 2 * H:] + r * gh[:, 2 * H:])
    h_new = (1.0 - z) * n + z * h_r[...]
    hn_r[...] = h_new
    psum_r[...] = jnp.sum(h_new, axis=0, keepdims=True)[None]
    psq_r[...] = jnp.sum(h_new * h_new, axis=0, keepdims=True)[None]


def _tc_final(S, y, degf, b1_2, b2_2, W_ih, b_ih2, gh, h2):
    return pl.pallas_call(
        _tc_final_body,
        grid=(GRID,),
        in_specs=[
            pl.BlockSpec((NSC, RB, HHALF), lambda i: (0, i, 0)),
            pl.BlockSpec((NSC, RB, HHALF), lambda i: (0, i, 0)),
            pl.BlockSpec((RB, 1), lambda i: (i, 0)),
            pl.BlockSpec((1, H), lambda i: (0, 0)),
            pl.BlockSpec((1, H), lambda i: (0, 0)),
            pl.BlockSpec((3 * H, H), lambda i: (0, 0)),
            pl.BlockSpec((1, 3 * H), lambda i: (0, 0)),
            pl.BlockSpec((1, 3 * H), lambda i: (0, 0)),
            pl.BlockSpec((1, H), lambda i: (0, 0)),
        ],
        out_specs=[
            pl.BlockSpec((RB, H), lambda i: (i, 0)),
            pl.BlockSpec((1, 1, H), lambda i: (i, 0, 0)),
            pl.BlockSpec((1, 1, H), lambda i: (i, 0, 0)),
        ],
        out_shape=[
            jax.ShapeDtypeStruct((N, H), jnp.float32),
            jax.ShapeDtypeStruct((GRID, 1, H), jnp.float32),
            jax.ShapeDtypeStruct((GRID, 1, H), jnp.float32),
        ],
    )(S, y, degf, b1_2, b2_2, W_ih, b_ih2, gh, h2)


def _tc_norm_body(hn_r, psum_r, psq_r, gamma_r, beta_r, out_r):
    tot = jnp.sum(psum_r[...], axis=0)
    tot2 = jnp.sum(psq_r[...], axis=0)
    mean = tot * (1.0 / N)
    var = tot2 * (1.0 / N) - mean * mean
    inv = gamma_r[...] * lax.rsqrt(var + 1e-5)
    out_r[...] = (hn_r[...] - mean) * inv + beta_r[...]


def _tc_norm(hn, psum, psq, gamma2, beta2):
    return pl.pallas_call(
        _tc_norm_body,
        grid=(GRID,),
        in_specs=[
            pl.BlockSpec((RB, H), lambda i: (i, 0)),
            pl.BlockSpec((GRID, 1, H), lambda i: (0, 0, 0)),
            pl.BlockSpec((GRID, 1, H), lambda i: (0, 0, 0)),
            pl.BlockSpec((1, H), lambda i: (0, 0)),
            pl.BlockSpec((1, H), lambda i: (0, 0)),
        ],
        out_specs=pl.BlockSpec((RB, H), lambda i: (i, 0)),
        out_shape=jax.ShapeDtypeStruct((N, H), jnp.float32),
    )(hn, psum, psq, gamma2, beta2)


def kernel(x, edge_index, h, W1, b1, W2, b2, W_ih, b_ih, W_hh, b_hh,
           gamma, beta):
    src = edge_index[0]
    dst = edge_index[1]
    pad = jnp.full((EPAD - E,), N, jnp.int32)
    srcp = jnp.concatenate([src, pad])
    dstp = jnp.concatenate([dst, pad])

    eye = jnp.eye(128, dtype=jnp.float32)
    degH = _sc_degree(eye, dstp.reshape(NSC * NSUB, -1, CHUNK))
    degf = (degH[0] + degH[1]).reshape(NPAD, 1)

    h2 = h.reshape(1, H)
    y, gh = _tc_prep(x, W1, W2, degf, h2, W_hh, b_hh.reshape(1, 3 * H))

    S = _sc_scatter(y.reshape(NSC * NPAD, HHALF),
                    srcp.reshape(NSUB, 2, -1, SCH),
                    dstp.reshape(NSUB, 2, -1, SCH))

    hn, psum, psq = _tc_final(S, y, degf, b1.reshape(1, H), b2.reshape(1, H),
                              W_ih, b_ih.reshape(1, 3 * H), gh, h2)
    return _tc_norm(hn, psum, psq, gamma.reshape(1, H), beta.reshape(1, H))
